# baseline (speedup 1.0000x reference)
import functools
import math

import jax
import jax.numpy as jnp
from jax import lax
from jax.experimental import pallas as pl
from jax.experimental.pallas import tpu as pltpu
from jax.experimental.pallas import tpu_sc as plsc

_NUM_CORES = 2
_NUM_SUBCORES = 16
_NUM_WORKERS = _NUM_CORES * _NUM_SUBCORES

_GATHER_WINDOW = 128

_TC_BLOCK = 8192


def _sc_gather(coeff_weight, flat_idx):
    n = flat_idx.shape[0]
    idx2d = flat_idx.reshape(1, n)
    n_bases = coeff_weight.shape[1]
    mesh = plsc.VectorSubcoreMesh(core_axis_name="c", subcore_axis_name="s")

    @functools.partial(
        pl.kernel,
        out_type=jax.ShapeDtypeStruct((n, n_bases), coeff_weight.dtype),
        mesh=mesh,
        compiler_params=pltpu.CompilerParams(use_tc_tiling_on_sc=False),
    )
    def gather_kernel(table_hbm, idx_hbm, out_hbm):
        def body(idx_vmem, out_vmem):
            pltpu.sync_copy(table_hbm.at[idx_vmem.at[0]], out_vmem)

        pltpu.emit_pipeline(
            body,
            grid=(n // _GATHER_WINDOW,),
            in_specs=[
                pl.BlockSpec((1, _GATHER_WINDOW), index_map=lambda i: (0, i))
            ],
            out_specs=[
                pl.BlockSpec((_GATHER_WINDOW, n_bases), index_map=lambda i: (i, 0))
            ],
            core_axis_name=("c", "s"),
            dimension_semantics=(pltpu.PARALLEL,),
        )(idx_hbm, out_hbm)

    return gather_kernel(coeff_weight, idx2d)


def _tc_combine_kernel(g_ref, b_ref, o_ref, *, scale):
    g = g_ref[...]
    m = jnp.max(g, axis=1, keepdims=True)
    e = jnp.exp(g - m)
    s = jnp.sum(e, axis=1, keepdims=True)
    p = e * (scale / s)
    b = b_ref[...]
    acc = p[:, 0:1] * b[0:1, :]
    for k in range(1, b.shape[0]):
        acc = acc + p[:, k : k + 1] * b[k : k + 1, :]
    o_ref[...] = acc


def _tc_combine(gathered, bases):
    n, n_bases = gathered.shape
    d_model = bases.shape[1]
    scale = math.sqrt(d_model)
    grid = (n // _TC_BLOCK,)
    return pl.pallas_call(
        functools.partial(_tc_combine_kernel, scale=scale),
        grid=grid,
        in_specs=[
            pl.BlockSpec((_TC_BLOCK, n_bases), lambda i: (i, 0)),
            pl.BlockSpec((n_bases, d_model), lambda i: (0, 0)),
        ],
        out_specs=pl.BlockSpec((_TC_BLOCK, d_model), lambda i: (i, 0)),
        out_shape=jax.ShapeDtypeStruct((n, d_model), jnp.float32),
    )(gathered, bases)


def kernel(x, coeff_weight, base_embed_weight, lang_bias):
    batch, seq = x.shape
    d_model = base_embed_weight.shape[1]
    flat_idx = x.reshape(batch * seq)
    gathered = _sc_gather(coeff_weight, flat_idx)
    bases = base_embed_weight + lang_bias
    out = _tc_combine(gathered, bases)
    return out.reshape(batch, seq, d_model)


# speedup vs baseline: 8.6030x; 8.6030x over previous
import functools
import math

import jax
import jax.numpy as jnp
from jax import lax
from jax.experimental import pallas as pl
from jax.experimental.pallas import tpu as pltpu
from jax.experimental.pallas import tpu_sc as plsc

_NUM_CORES = 2
_NUM_SUBCORES = 16
_NUM_WORKERS = _NUM_CORES * _NUM_SUBCORES

_WINDOW = 128
_HALF = 100
_GROUP = 10

_TC_SEQ_BLOCK = 8


def _sc_gather_planes(table_t, idx_rows):
    n_bases, _ = table_t.shape
    nwin = idx_rows.shape[0]
    per_worker = nwin // _NUM_WORKERS
    n_halves = per_worker // _HALF
    mesh = plsc.VectorSubcoreMesh(core_axis_name="c", subcore_axis_name="s")
    plane_ty = jax.ShapeDtypeStruct((nwin, _WINDOW), jnp.float32)

    @functools.partial(
        pl.kernel,
        out_type=[plane_ty] * n_bases,
        mesh=mesh,
        scratch_types=[
            pltpu.VMEM((_HALF, _WINDOW), jnp.int32),
            pltpu.VMEM((n_bases, _HALF, _WINDOW), jnp.float32),
            pltpu.SemaphoreType.DMA,
            pltpu.SemaphoreType.DMA,
        ],
        compiler_params=pltpu.CompilerParams(use_tc_tiling_on_sc=False),
    )
    def gather_kernel(table_hbm, idx_hbm, *rest):
        outs = rest[:n_bases]
        idx_v, stage, sem_g, sem_o = rest[n_bases:]
        wid = lax.axis_index("s") * _NUM_CORES + lax.axis_index("c")
        base = wid * per_worker

        def fire(w):
            for j in range(_GROUP):
                for k in range(n_bases):
                    pltpu.async_copy(
                        table_hbm.at[k].at[idx_v.at[w + j]],
                        stage.at[k].at[w + j],
                        sem_g,
                    )

        def drain(w):
            for j in range(_GROUP):
                for k in range(n_bases):
                    pltpu.make_async_copy(
                        table_hbm.at[k].at[idx_v.at[w + j]],
                        stage.at[k].at[w + j],
                        sem_g,
                    ).wait()

        for half in range(n_halves):
            row0 = base + half * _HALF
            pltpu.sync_copy(idx_hbm.at[pl.ds(row0, _HALF)], idx_v)

            @pl.loop(0, _HALF + _GROUP, step=_GROUP)
            def _(w):
                @pl.when(w < _HALF)
                def _():
                    fire(w)

                @pl.when(w >= _GROUP)
                def _():
                    drain(w - _GROUP)

            for k in range(n_bases):
                pltpu.async_copy(stage.at[k], outs[k].at[pl.ds(row0, _HALF)], sem_o)
            for k in range(n_bases):
                pltpu.make_async_copy(
                    stage.at[k], outs[k].at[pl.ds(row0, _HALF)], sem_o
                ).wait()

    return gather_kernel(table_t, idx_rows)


def _tc_combine_kernel(g0_ref, g1_ref, g2_ref, g3_ref, bt_ref, lt_ref, o_ref):
    b = (bt_ref[...] + lt_ref[...]) * 8.0
    for r in range(o_ref.shape[0]):
        rows = [g[r : r + 1, :] for g in (g0_ref, g1_ref, g2_ref, g3_ref)]
        m = jnp.maximum(jnp.maximum(rows[0][...], rows[1][...]),
                        jnp.maximum(rows[2][...], rows[3][...]))
        es = [jnp.exp(g[...] - m) for g in rows]
        inv = 1.0 / (es[0] + es[1] + es[2] + es[3])
        acc = (b[:, 0:1] * (es[0] * inv) + b[:, 1:2] * (es[1] * inv)
               + b[:, 2:3] * (es[2] * inv) + b[:, 3:4] * (es[3] * inv))
        o_ref[r] = acc


def _tc_combine(planes, bases_t, lang_bias_t, seq, batch):
    d_model = bases_t.shape[0]
    grid = (seq // _TC_SEQ_BLOCK,)
    plane_spec = pl.BlockSpec((_TC_SEQ_BLOCK, batch), lambda i: (i, 0))
    small_spec = pl.BlockSpec(bases_t.shape, lambda i: (0, 0))
    return pl.pallas_call(
        _tc_combine_kernel,
        grid=grid,
        in_specs=[plane_spec] * 4 + [small_spec, small_spec],
        out_specs=pl.BlockSpec((_TC_SEQ_BLOCK, d_model, batch), lambda i: (i, 0, 0)),
        out_shape=jax.ShapeDtypeStruct((seq, d_model, batch), jnp.float32),
    )(*planes, bases_t, lang_bias_t)


def kernel(x, coeff_weight, base_embed_weight, lang_bias):
    batch, seq = x.shape
    d_model = base_embed_weight.shape[1]
    n = batch * seq

    table_t = coeff_weight.T
    idx_rows = x.T.reshape(n // _WINDOW, _WINDOW)

    planes_rows = _sc_gather_planes(table_t, idx_rows)
    planes = [p.reshape(seq, batch) for p in planes_rows]

    out_phys = _tc_combine(
        planes, base_embed_weight.T, lang_bias.T, seq, batch
    )
    return jnp.transpose(out_phys, (2, 0, 1))
